# Initial kernel scaffold; baseline (speedup 1.0000x reference)
import jax
import jax.numpy as jnp
from jax.experimental import pallas as pl


def kernel(x, edge_index, batch, s, W1, W2, gamma, beta, Wg, bg, Wh, bh):
    raise NotImplementedError("write your pallas kernel here")


# speedup vs baseline: 5.6973x; 5.6973x over previous
import functools

import jax
import jax.numpy as jnp
from jax import lax
from jax.experimental import pallas as pl
from jax.experimental.pallas import tpu as pltpu
from jax.experimental.pallas import tpu_sc as plsc

N = 10000
E = 320000
D = 128
L = 3
G = 128

NTILES = 16
NCORES = 2
WIN = 128
ROWS_PER_TILE = 632
N_PAD = NTILES * ROWS_PER_TILE
NWIN = 160
E_PAD = NTILES * NWIN * WIN
ICH = 16
NWIN_DEG = 80
E_DEG_PAD = NCORES * NTILES * NWIN_DEG * WIN
DEG_W = 16

BLK = 1000
NBLK = N // BLK

_MM = dict(preferred_element_type=jnp.float32, precision=lax.Precision.HIGHEST)

@functools.cache
def _vec_mesh():
    return plsc.VectorSubcoreMesh(core_axis_name="c", subcore_axis_name="s",
                                  num_cores=NCORES, num_subcores=NTILES)


def _zero_buf(buf, rows, cols):
    zv = jnp.zeros((16,), jnp.float32)

    @pl.loop(0, rows)
    def _(r):
        @pl.loop(0, cols, step=16)
        def _(c):
            buf[r, pl.ds(c, 16)] = zv


def _fill_ones(buf, rows, cols):
    ov = jnp.ones((16,), jnp.float32)

    @pl.loop(0, rows)
    def _(r):
        @pl.loop(0, cols, step=16)
        def _(c):
            buf[r, pl.ds(c, 16)] = ov


def _zero_acc_slice(zbuf, acc, base):
    for k in range(4):
        pltpu.sync_copy(zbuf, acc.at[pl.ds(base + 128 * k, 128)])
    pltpu.sync_copy(zbuf.at[pl.ds(0, ROWS_PER_TILE - 512)],
                    acc.at[pl.ds(base + 512, ROWS_PER_TILE - 512)])


def _writeout_acc_slice(acc, buf, out_hbm, cid, base):
    for k in range(4):
        pltpu.sync_copy(acc.at[pl.ds(base + 128 * k, 128)], buf)
        pltpu.sync_copy(buf, out_hbm.at[cid, pl.ds(base + 128 * k, 128)])
    rem = ROWS_PER_TILE - 512
    pltpu.sync_copy(acc.at[pl.ds(base + 512, rem)], buf.at[pl.ds(0, rem)])
    pltpu.sync_copy(buf.at[pl.ds(0, rem)], out_hbm.at[cid, pl.ds(base + 512, rem)])


@functools.cache
def _make_sc_deg():
    return functools.partial(
        pl.kernel,
        out_type=jax.ShapeDtypeStruct((NCORES, N_PAD, DEG_W), jnp.float32),
        mesh=_vec_mesh(),
        scratch_types=[
            pltpu.VMEM((NWIN_DEG, WIN), jnp.int32),
            pltpu.VMEM((WIN, DEG_W), jnp.float32),
            pltpu.VMEM((128, DEG_W), jnp.float32),
            pltpu.VMEM_SHARED((N_PAD, DEG_W), jnp.float32),
        ],
    )(_sc_deg_body)


def _sc_deg(dstd):
    return _make_sc_deg()(dstd)


def _sc_deg_body(dstd_hbm, out_hbm, dst_v, ones_v, zbuf, acc):
    cid = lax.axis_index("c")
    tid = lax.axis_index("s")
    base = tid * ROWS_PER_TILE
    pltpu.sync_copy(dstd_hbm.at[cid, tid], dst_v)
    _fill_ones(ones_v, WIN, DEG_W)
    _zero_buf(zbuf, 128, DEG_W)
    _zero_acc_slice(zbuf, acc, base)
    plsc.subcore_barrier()

    @pl.loop(0, NWIN_DEG)
    def _(w):
        pltpu.sync_copy(ones_v, acc.at[dst_v.at[w]], add=True)

    plsc.subcore_barrier()
    _writeout_acc_slice(acc, zbuf, out_hbm, cid, base)


@functools.cache
def _make_sc_segsum():
    return functools.partial(
        pl.kernel,
        out_type=jax.ShapeDtypeStruct((NCORES, N_PAD, D), jnp.float32),
        mesh=_vec_mesh(),
        scratch_types=[
            pltpu.VMEM((ICH, WIN), jnp.int32),
            pltpu.VMEM((ICH, WIN), jnp.int32),
            pltpu.VMEM((WIN, D), jnp.float32),
            pltpu.VMEM_SHARED((N_PAD, D), jnp.float32),
        ],
    )(_sc_segsum_body)


def _sc_segsum(Y, SWP, srcw, dstw):
    return _make_sc_segsum()(Y, SWP, srcw, dstw)


def _sc_segsum_body(y_hbm, swp_hbm, srcw_hbm, dstw_hbm, out_hbm, src_v, dst_v,
                    buf, acc):
    cid = lax.axis_index("c")
    tid = lax.axis_index("s")
    base = tid * ROWS_PER_TILE
    _zero_buf(buf, WIN, D)
    _zero_acc_slice(buf, acc, base)
    plsc.subcore_barrier()

    @pl.loop(0, NWIN, step=ICH)
    def _(c0):
        pltpu.sync_copy(srcw_hbm.at[tid, pl.ds(c0, ICH)], src_v)
        pltpu.sync_copy(dstw_hbm.at[tid, pl.ds(c0, ICH)], dst_v)

        @pl.loop(0, ICH)
        def _(w):
            @pl.when(cid == 0)
            def _():
                pltpu.sync_copy(y_hbm.at[src_v.at[w]], buf)

            @pl.when(cid == 1)
            def _():
                pltpu.sync_copy(swp_hbm.at[src_v.at[w]], buf)

            pltpu.sync_copy(buf, acc.at[dst_v.at[w]], add=True)

    plsc.subcore_barrier()
    _writeout_acc_slice(acc, buf, out_hbm, cid, base)


def _tc0_body(x_ref, s_ref, deg_ref, w1_ref, wg_ref, y_ref, swp_ref, dinv_ref):
    deg = deg_ref[0, :, 0] + deg_ref[1, :, 0] + 1.0
    dinv = jax.lax.rsqrt(deg)[:, None]
    xb = x_ref[...]
    sb = s_ref[...]
    y_ref[...] = (jax.lax.dot_general(xb, w1_ref[:D], (((1,), (0,)), ((), ())), **_MM)
                  + jax.lax.dot_general(sb, w1_ref[D:], (((1,), (0,)), ((), ())), **_MM))
    sw = jax.lax.dot_general(sb, wg_ref[...], (((1,), (0,)), ((), ())), **_MM)
    swp_ref[...] = dinv * sw
    dinv_ref[...] = dinv


def _tc_layer0(x, s, degout, W1_0, Wg_0):
    return pl.pallas_call(
        _tc0_body,
        grid=(NBLK,),
        in_specs=[
            pl.BlockSpec((BLK, D), lambda j: (j, 0)),
            pl.BlockSpec((BLK, D), lambda j: (j, 0)),
            pl.BlockSpec((NCORES, BLK, DEG_W), lambda j: (0, j, 0)),
            pl.BlockSpec((2 * D, D), lambda j: (0, 0)),
            pl.BlockSpec((D, D), lambda j: (0, 0)),
        ],
        out_specs=[
            pl.BlockSpec((BLK, D), lambda j: (j, 0)),
            pl.BlockSpec((BLK, D), lambda j: (j, 0)),
            pl.BlockSpec((BLK, 1), lambda j: (j, 0)),
        ],
        out_shape=[
            jax.ShapeDtypeStruct((N, D), jnp.float32),
            jax.ShapeDtypeStruct((N, D), jnp.float32),
            jax.ShapeDtypeStruct((N, 1), jnp.float32),
        ],
    )(x, s, degout, W1_0, Wg_0)


def _tc_mid_body(y_ref, swp_ref, agg_ref, dinv_ref, w2_ref, bg_ref, w1n_ref,
                 wgn_ref, yn_ref, swpn_ref):
    y = y_ref[...]
    swp = swp_ref[...]
    aggy = agg_ref[0]
    aggs = agg_ref[1]
    dinv = dinv_ref[...]
    h = y + aggy
    h = jnp.where(h >= 0, h, 0.01 * h)
    xn = jax.lax.dot_general(h, w2_ref[...], (((1,), (0,)), ((), ())), **_MM)
    sn = jnp.tanh(dinv * (aggs + swp) + bg_ref[...])
    yn_ref[...] = (jax.lax.dot_general(xn, w1n_ref[:D], (((1,), (0,)), ((), ())), **_MM)
                   + jax.lax.dot_general(sn, w1n_ref[D:], (((1,), (0,)), ((), ())), **_MM))
    swn = jax.lax.dot_general(sn, wgn_ref[...], (((1,), (0,)), ((), ())), **_MM)
    swpn_ref[...] = dinv * swn


def _tc_mid(Y, SWP, agg, dinv, W2_i, bg_i, W1_n, Wg_n):
    return pl.pallas_call(
        _tc_mid_body,
        grid=(NBLK,),
        in_specs=[
            pl.BlockSpec((BLK, D), lambda j: (j, 0)),
            pl.BlockSpec((BLK, D), lambda j: (j, 0)),
            pl.BlockSpec((NCORES, BLK, D), lambda j: (0, j, 0)),
            pl.BlockSpec((BLK, 1), lambda j: (j, 0)),
            pl.BlockSpec((D, D), lambda j: (0, 0)),
            pl.BlockSpec((1, D), lambda j: (0, 0)),
            pl.BlockSpec((2 * D, D), lambda j: (0, 0)),
            pl.BlockSpec((D, D), lambda j: (0, 0)),
        ],
        out_specs=[
            pl.BlockSpec((BLK, D), lambda j: (j, 0)),
            pl.BlockSpec((BLK, D), lambda j: (j, 0)),
        ],
        out_shape=[
            jax.ShapeDtypeStruct((N, D), jnp.float32),
            jax.ShapeDtypeStruct((N, D), jnp.float32),
        ],
    )(Y, SWP, agg, dinv, W2_i, bg_i, W1_n, Wg_n)


def _tc_last_body(y_ref, swp_ref, agg_ref, dinv_ref, w2_ref, bg_ref,
                  x2_ref, s2_ref, s1_ref, ss_ref):
    j = pl.program_id(0)
    y = y_ref[...]
    swp = swp_ref[...]
    h = y + agg_ref[0]
    h = jnp.where(h >= 0, h, 0.01 * h)
    xn = jax.lax.dot_general(h, w2_ref[...], (((1,), (0,)), ((), ())), **_MM)
    sn = jnp.tanh(dinv_ref[...] * (agg_ref[1] + swp) + bg_ref[...])
    x2_ref[...] = xn
    s2_ref[...] = sn
    s1 = jnp.sum(xn, axis=0, keepdims=True)
    ss = jnp.sum(xn * xn, axis=0, keepdims=True)

    @pl.when(j == 0)
    def _():
        s1_ref[...] = s1
        ss_ref[...] = ss

    @pl.when(j != 0)
    def _():
        s1_ref[...] += s1
        ss_ref[...] += ss


def _tc_last(Y, SWP, agg, dinv, W2_i, bg_i):
    return pl.pallas_call(
        _tc_last_body,
        grid=(NBLK,),
        in_specs=[
            pl.BlockSpec((BLK, D), lambda j: (j, 0)),
            pl.BlockSpec((BLK, D), lambda j: (j, 0)),
            pl.BlockSpec((NCORES, BLK, D), lambda j: (0, j, 0)),
            pl.BlockSpec((BLK, 1), lambda j: (j, 0)),
            pl.BlockSpec((D, D), lambda j: (0, 0)),
            pl.BlockSpec((1, D), lambda j: (0, 0)),
        ],
        out_specs=[
            pl.BlockSpec((BLK, D), lambda j: (j, 0)),
            pl.BlockSpec((BLK, D), lambda j: (j, 0)),
            pl.BlockSpec((1, D), lambda j: (0, 0)),
            pl.BlockSpec((1, D), lambda j: (0, 0)),
        ],
        out_shape=[
            jax.ShapeDtypeStruct((N, D), jnp.float32),
            jax.ShapeDtypeStruct((N, D), jnp.float32),
            jax.ShapeDtypeStruct((1, D), jnp.float32),
            jax.ShapeDtypeStruct((1, D), jnp.float32),
        ],
        compiler_params=pltpu.CompilerParams(
            dimension_semantics=("arbitrary",)),
    )(Y, SWP, agg, dinv, W2_i, bg_i)


def _tc_final_body(x2_ref, s2_ref, b_ref, s1_ref, ss_ref, gamma_ref, beta_ref,
                   wh_ref, bh_ref, xl_ref, pooled_ref):
    j = pl.program_id(0)
    mean = s1_ref[...] / N
    var = ss_ref[...] / N - mean * mean
    inv = jax.lax.rsqrt(var + 1e-4)
    xl = (x2_ref[...] - mean) * inv * gamma_ref[...] + beta_ref[...]
    xl_ref[...] = xl
    o = (jax.lax.dot_general(xl, wh_ref[:D], (((1,), (0,)), ((), ())), **_MM)
         + jax.lax.dot_general(s2_ref[...], wh_ref[D:], (((1,), (0,)), ((), ())), **_MM)
         + bh_ref[...])
    gid = jax.lax.broadcasted_iota(jnp.int32, (BLK, G), 1)
    onehot = (b_ref[...] == gid).astype(jnp.float32)
    part = jax.lax.dot_general(onehot, o, (((0,), (0,)), ((), ())), **_MM)

    @pl.when(j == 0)
    def _():
        pooled_ref[...] = part

    @pl.when(j != 0)
    def _():
        pooled_ref[...] += part


def _tc_final(X2, S2, batch, S1, SS, gamma2, beta2, Wh, bh):
    return pl.pallas_call(
        _tc_final_body,
        grid=(NBLK,),
        in_specs=[
            pl.BlockSpec((BLK, D), lambda j: (j, 0)),
            pl.BlockSpec((BLK, D), lambda j: (j, 0)),
            pl.BlockSpec((BLK, 1), lambda j: (j, 0)),
            pl.BlockSpec((1, D), lambda j: (0, 0)),
            pl.BlockSpec((1, D), lambda j: (0, 0)),
            pl.BlockSpec((1, D), lambda j: (0, 0)),
            pl.BlockSpec((1, D), lambda j: (0, 0)),
            pl.BlockSpec((2 * D, D), lambda j: (0, 0)),
            pl.BlockSpec((1, D), lambda j: (0, 0)),
        ],
        out_specs=[
            pl.BlockSpec((BLK, D), lambda j: (j, 0)),
            pl.BlockSpec((G, D), lambda j: (0, 0)),
        ],
        out_shape=[
            jax.ShapeDtypeStruct((N, D), jnp.float32),
            jax.ShapeDtypeStruct((G, D), jnp.float32),
        ],
        compiler_params=pltpu.CompilerParams(
            dimension_semantics=("arbitrary",)),
    )(X2, S2, batch, S1, SS, gamma2, beta2, Wh, bh)


def kernel(x, edge_index, batch, s, W1, W2, gamma, beta, Wg, bg, Wh, bh):
    src = edge_index[0].astype(jnp.int32)
    dst = edge_index[1].astype(jnp.int32)
    srcw = jnp.concatenate(
        [src, jnp.zeros((E_PAD - E,), jnp.int32)]).reshape(NTILES, NWIN, WIN)
    dstw = jnp.concatenate(
        [dst, jnp.full((E_PAD - E,), N_PAD - 1, jnp.int32)]
    ).reshape(NTILES, NWIN, WIN)
    dstd = jnp.concatenate(
        [dst, jnp.full((E_DEG_PAD - E,), N_PAD - 1, jnp.int32)]
    ).reshape(NCORES, NTILES, NWIN_DEG, WIN)

    degout = _sc_deg(dstd)
    Y, SWP, dinv = _tc_layer0(x, s, degout, W1[0], Wg[0])
    X2 = S2 = S1 = SS = None
    for i in range(L):
        agg = _sc_segsum(Y, SWP, srcw, dstw)
        if i < L - 1:
            Y, SWP = _tc_mid(Y, SWP, agg, dinv, W2[i], bg[i].reshape(1, D),
                             W1[i + 1], Wg[i + 1])
        else:
            X2, S2, S1, SS = _tc_last(Y, SWP, agg, dinv, W2[i],
                                      bg[i].reshape(1, D))
    x_local, pooled = _tc_final(X2, S2, batch.astype(jnp.int32).reshape(N, 1),
                                S1, SS, gamma[L - 1].reshape(1, D),
                                beta[L - 1].reshape(1, D), Wh,
                                bh.reshape(1, D))
    return (pooled, x_local)


# speedup vs baseline: 6.4455x; 1.1313x over previous
import functools

import jax
import jax.numpy as jnp
from jax import lax
from jax.experimental import pallas as pl
from jax.experimental.pallas import tpu as pltpu
from jax.experimental.pallas import tpu_sc as plsc

N = 10000
E = 320000
D = 128
L = 3
G = 128

NTILES = 16
NCORES = 2
WIN = 128
ROWS_PER_TILE = 632
N_PAD = NTILES * ROWS_PER_TILE
NWIN = 160
E_PAD = NTILES * NWIN * WIN
ICH = 40
NWIN_DEG = 80
E_DEG_PAD = NCORES * NTILES * NWIN_DEG * WIN
DEG_W = 16

BLK = 1000
NBLK = N // BLK

_MM = dict(preferred_element_type=jnp.float32, precision=lax.Precision.HIGHEST)

@functools.cache
def _vec_mesh():
    return plsc.VectorSubcoreMesh(core_axis_name="c", subcore_axis_name="s",
                                  num_cores=NCORES, num_subcores=NTILES)


def _zero_buf(buf, rows, cols):
    zv = jnp.zeros((16,), jnp.float32)

    @pl.loop(0, rows)
    def _(r):
        @pl.loop(0, cols, step=16)
        def _(c):
            buf[r, pl.ds(c, 16)] = zv


def _fill_ones(buf, rows, cols):
    ov = jnp.ones((16,), jnp.float32)

    @pl.loop(0, rows)
    def _(r):
        @pl.loop(0, cols, step=16)
        def _(c):
            buf[r, pl.ds(c, 16)] = ov


def _zero_acc_slice(zbuf, acc, base):
    for k in range(4):
        pltpu.sync_copy(zbuf, acc.at[pl.ds(base + 128 * k, 128)])
    pltpu.sync_copy(zbuf.at[pl.ds(0, ROWS_PER_TILE - 512)],
                    acc.at[pl.ds(base + 512, ROWS_PER_TILE - 512)])


def _writeout_acc_slice(acc, buf, out_hbm, cid, base):
    for k in range(4):
        pltpu.sync_copy(acc.at[pl.ds(base + 128 * k, 128)], buf)
        pltpu.sync_copy(buf, out_hbm.at[cid, pl.ds(base + 128 * k, 128)])
    rem = ROWS_PER_TILE - 512
    pltpu.sync_copy(acc.at[pl.ds(base + 512, rem)], buf.at[pl.ds(0, rem)])
    pltpu.sync_copy(buf.at[pl.ds(0, rem)], out_hbm.at[cid, pl.ds(base + 512, rem)])


@functools.cache
def _make_sc_deg():
    return functools.partial(
        pl.kernel,
        out_type=jax.ShapeDtypeStruct((NCORES, N_PAD, DEG_W), jnp.float32),
        mesh=_vec_mesh(),
        scratch_types=[
            pltpu.VMEM((NWIN_DEG, WIN), jnp.int32),
            pltpu.VMEM((WIN, DEG_W), jnp.float32),
            pltpu.VMEM((128, DEG_W), jnp.float32),
            pltpu.VMEM_SHARED((N_PAD, DEG_W), jnp.float32),
        ],
    )(_sc_deg_body)


def _sc_deg(dstd):
    return _make_sc_deg()(dstd)


def _sc_deg_body(dstd_hbm, out_hbm, dst_v, ones_v, zbuf, acc):
    cid = lax.axis_index("c")
    tid = lax.axis_index("s")
    base = tid * ROWS_PER_TILE
    pltpu.sync_copy(dstd_hbm.at[cid, tid], dst_v)
    _fill_ones(ones_v, WIN, DEG_W)
    _zero_buf(zbuf, 128, DEG_W)
    _zero_acc_slice(zbuf, acc, base)
    plsc.subcore_barrier()

    @pl.loop(0, NWIN_DEG)
    def _(w):
        pltpu.sync_copy(ones_v, acc.at[dst_v.at[w]], add=True)

    plsc.subcore_barrier()
    _writeout_acc_slice(acc, zbuf, out_hbm, cid, base)


@functools.cache
def _make_sc_segsum():
    return functools.partial(
        pl.kernel,
        out_type=jax.ShapeDtypeStruct((NCORES, N_PAD, D), jnp.float32),
        mesh=_vec_mesh(),
        scratch_types=[
            pltpu.VMEM((ICH, WIN), jnp.int32),
            pltpu.VMEM((ICH, WIN), jnp.int32),
            pltpu.VMEM((WIN, D), jnp.float32),
            pltpu.VMEM((WIN, D), jnp.float32),
            pltpu.SemaphoreType.DMA,
            pltpu.SemaphoreType.DMA,
            pltpu.SemaphoreType.DMA,
            pltpu.SemaphoreType.DMA,
            pltpu.VMEM_SHARED((N_PAD, D), jnp.float32),
        ],
    )(_sc_segsum_body)


def _sc_segsum(Y, SWP, srcw, dstw):
    return _make_sc_segsum()(Y, SWP, srcw, dstw)


def _sc_segsum_body(y_hbm, swp_hbm, srcw_hbm, dstw_hbm, out_hbm, src_v, dst_v,
                    bufA, bufB, gsA, gsB, ssA, ssB, acc):
    cid = lax.axis_index("c")
    tid = lax.axis_index("s")
    base = tid * ROWS_PER_TILE
    _zero_buf(bufA, WIN, D)
    _zero_acc_slice(bufA, acc, base)
    plsc.subcore_barrier()

    def _gather(idx_row, buf, sem):
        @pl.when(cid == 0)
        def _():
            pltpu.async_copy(y_hbm.at[idx_row], buf, sem)

        @pl.when(cid == 1)
        def _():
            pltpu.async_copy(swp_hbm.at[idx_row], buf, sem)

    def _gather_wait(buf, sem):
        pltpu.make_async_copy(y_hbm.at[src_v.at[0]], buf, sem).wait()

    def _scatter(buf, idx_row, sem):
        pltpu.async_copy(buf, acc.at[idx_row], sem, add=True)

    def _scatter_wait(buf, sem):
        pltpu.make_async_copy(buf, acc.at[dst_v.at[0]], sem).wait()

    @pl.loop(0, NWIN, step=ICH)
    def _(c0):
        pltpu.sync_copy(srcw_hbm.at[tid, pl.ds(c0, ICH)], src_v)
        pltpu.sync_copy(dstw_hbm.at[tid, pl.ds(c0, ICH)], dst_v)
        _gather(src_v.at[0], bufA, gsA)
        _gather(src_v.at[1], bufB, gsB)

        @pl.loop(2, ICH, step=2)
        def _(w):
            _gather_wait(bufA, gsA)
            _scatter(bufA, dst_v.at[w - 2], ssA)
            _gather_wait(bufB, gsB)
            _scatter(bufB, dst_v.at[w - 1], ssB)
            _scatter_wait(bufA, ssA)
            _gather(src_v.at[w], bufA, gsA)
            _scatter_wait(bufB, ssB)
            _gather(src_v.at[w + 1], bufB, gsB)

        _gather_wait(bufA, gsA)
        _scatter(bufA, dst_v.at[ICH - 2], ssA)
        _gather_wait(bufB, gsB)
        _scatter(bufB, dst_v.at[ICH - 1], ssB)
        _scatter_wait(bufA, ssA)
        _scatter_wait(bufB, ssB)

    plsc.subcore_barrier()
    _writeout_acc_slice(acc, bufA, out_hbm, cid, base)


def _tc0_body(x_ref, s_ref, deg_ref, w1_ref, wg_ref, y_ref, swp_ref, dinv_ref):
    deg = deg_ref[0, :, 0] + deg_ref[1, :, 0] + 1.0
    dinv = jax.lax.rsqrt(deg)[:, None]
    xb = x_ref[...]
    sb = s_ref[...]
    y_ref[...] = (jax.lax.dot_general(xb, w1_ref[:D], (((1,), (0,)), ((), ())), **_MM)
                  + jax.lax.dot_general(sb, w1_ref[D:], (((1,), (0,)), ((), ())), **_MM))
    sw = jax.lax.dot_general(sb, wg_ref[...], (((1,), (0,)), ((), ())), **_MM)
    swp_ref[...] = dinv * sw
    dinv_ref[...] = dinv


def _tc_layer0(x, s, degout, W1_0, Wg_0):
    return pl.pallas_call(
        _tc0_body,
        grid=(NBLK,),
        in_specs=[
            pl.BlockSpec((BLK, D), lambda j: (j, 0)),
            pl.BlockSpec((BLK, D), lambda j: (j, 0)),
            pl.BlockSpec((NCORES, BLK, DEG_W), lambda j: (0, j, 0)),
            pl.BlockSpec((2 * D, D), lambda j: (0, 0)),
            pl.BlockSpec((D, D), lambda j: (0, 0)),
        ],
        out_specs=[
            pl.BlockSpec((BLK, D), lambda j: (j, 0)),
            pl.BlockSpec((BLK, D), lambda j: (j, 0)),
            pl.BlockSpec((BLK, 1), lambda j: (j, 0)),
        ],
        out_shape=[
            jax.ShapeDtypeStruct((N, D), jnp.float32),
            jax.ShapeDtypeStruct((N, D), jnp.float32),
            jax.ShapeDtypeStruct((N, 1), jnp.float32),
        ],
    )(x, s, degout, W1_0, Wg_0)


def _tc_mid_body(y_ref, swp_ref, agg_ref, dinv_ref, w2_ref, bg_ref, w1n_ref,
                 wgn_ref, yn_ref, swpn_ref):
    y = y_ref[...]
    swp = swp_ref[...]
    aggy = agg_ref[0]
    aggs = agg_ref[1]
    dinv = dinv_ref[...]
    h = y + aggy
    h = jnp.where(h >= 0, h, 0.01 * h)
    xn = jax.lax.dot_general(h, w2_ref[...], (((1,), (0,)), ((), ())), **_MM)
    sn = jnp.tanh(dinv * (aggs + swp) + bg_ref[...])
    yn_ref[...] = (jax.lax.dot_general(xn, w1n_ref[:D], (((1,), (0,)), ((), ())), **_MM)
                   + jax.lax.dot_general(sn, w1n_ref[D:], (((1,), (0,)), ((), ())), **_MM))
    swn = jax.lax.dot_general(sn, wgn_ref[...], (((1,), (0,)), ((), ())), **_MM)
    swpn_ref[...] = dinv * swn


def _tc_mid(Y, SWP, agg, dinv, W2_i, bg_i, W1_n, Wg_n):
    return pl.pallas_call(
        _tc_mid_body,
        grid=(NBLK,),
        in_specs=[
            pl.BlockSpec((BLK, D), lambda j: (j, 0)),
            pl.BlockSpec((BLK, D), lambda j: (j, 0)),
            pl.BlockSpec((NCORES, BLK, D), lambda j: (0, j, 0)),
            pl.BlockSpec((BLK, 1), lambda j: (j, 0)),
            pl.BlockSpec((D, D), lambda j: (0, 0)),
            pl.BlockSpec((1, D), lambda j: (0, 0)),
            pl.BlockSpec((2 * D, D), lambda j: (0, 0)),
            pl.BlockSpec((D, D), lambda j: (0, 0)),
        ],
        out_specs=[
            pl.BlockSpec((BLK, D), lambda j: (j, 0)),
            pl.BlockSpec((BLK, D), lambda j: (j, 0)),
        ],
        out_shape=[
            jax.ShapeDtypeStruct((N, D), jnp.float32),
            jax.ShapeDtypeStruct((N, D), jnp.float32),
        ],
    )(Y, SWP, agg, dinv, W2_i, bg_i, W1_n, Wg_n)


def _tc_last_body(y_ref, swp_ref, agg_ref, dinv_ref, w2_ref, bg_ref,
                  x2_ref, s2_ref, s1_ref, ss_ref):
    j = pl.program_id(0)
    y = y_ref[...]
    swp = swp_ref[...]
    h = y + agg_ref[0]
    h = jnp.where(h >= 0, h, 0.01 * h)
    xn = jax.lax.dot_general(h, w2_ref[...], (((1,), (0,)), ((), ())), **_MM)
    sn = jnp.tanh(dinv_ref[...] * (agg_ref[1] + swp) + bg_ref[...])
    x2_ref[...] = xn
    s2_ref[...] = sn
    s1 = jnp.sum(xn, axis=0, keepdims=True)
    ss = jnp.sum(xn * xn, axis=0, keepdims=True)

    @pl.when(j == 0)
    def _():
        s1_ref[...] = s1
        ss_ref[...] = ss

    @pl.when(j != 0)
    def _():
        s1_ref[...] += s1
        ss_ref[...] += ss


def _tc_last(Y, SWP, agg, dinv, W2_i, bg_i):
    return pl.pallas_call(
        _tc_last_body,
        grid=(NBLK,),
        in_specs=[
            pl.BlockSpec((BLK, D), lambda j: (j, 0)),
            pl.BlockSpec((BLK, D), lambda j: (j, 0)),
            pl.BlockSpec((NCORES, BLK, D), lambda j: (0, j, 0)),
            pl.BlockSpec((BLK, 1), lambda j: (j, 0)),
            pl.BlockSpec((D, D), lambda j: (0, 0)),
            pl.BlockSpec((1, D), lambda j: (0, 0)),
        ],
        out_specs=[
            pl.BlockSpec((BLK, D), lambda j: (j, 0)),
            pl.BlockSpec((BLK, D), lambda j: (j, 0)),
            pl.BlockSpec((1, D), lambda j: (0, 0)),
            pl.BlockSpec((1, D), lambda j: (0, 0)),
        ],
        out_shape=[
            jax.ShapeDtypeStruct((N, D), jnp.float32),
            jax.ShapeDtypeStruct((N, D), jnp.float32),
            jax.ShapeDtypeStruct((1, D), jnp.float32),
            jax.ShapeDtypeStruct((1, D), jnp.float32),
        ],
        compiler_params=pltpu.CompilerParams(
            dimension_semantics=("arbitrary",)),
    )(Y, SWP, agg, dinv, W2_i, bg_i)


def _tc_final_body(x2_ref, s2_ref, b_ref, s1_ref, ss_ref, gamma_ref, beta_ref,
                   wh_ref, bh_ref, xl_ref, pooled_ref):
    j = pl.program_id(0)
    mean = s1_ref[...] / N
    var = ss_ref[...] / N - mean * mean
    inv = jax.lax.rsqrt(var + 1e-4)
    xl = (x2_ref[...] - mean) * inv * gamma_ref[...] + beta_ref[...]
    xl_ref[...] = xl
    o = (jax.lax.dot_general(xl, wh_ref[:D], (((1,), (0,)), ((), ())), **_MM)
         + jax.lax.dot_general(s2_ref[...], wh_ref[D:], (((1,), (0,)), ((), ())), **_MM)
         + bh_ref[...])
    gid = jax.lax.broadcasted_iota(jnp.int32, (BLK, G), 1)
    onehot = (b_ref[...] == gid).astype(jnp.float32)
    part = jax.lax.dot_general(onehot, o, (((0,), (0,)), ((), ())), **_MM)

    @pl.when(j == 0)
    def _():
        pooled_ref[...] = part

    @pl.when(j != 0)
    def _():
        pooled_ref[...] += part


def _tc_final(X2, S2, batch, S1, SS, gamma2, beta2, Wh, bh):
    return pl.pallas_call(
        _tc_final_body,
        grid=(NBLK,),
        in_specs=[
            pl.BlockSpec((BLK, D), lambda j: (j, 0)),
            pl.BlockSpec((BLK, D), lambda j: (j, 0)),
            pl.BlockSpec((BLK, 1), lambda j: (j, 0)),
            pl.BlockSpec((1, D), lambda j: (0, 0)),
            pl.BlockSpec((1, D), lambda j: (0, 0)),
            pl.BlockSpec((1, D), lambda j: (0, 0)),
            pl.BlockSpec((1, D), lambda j: (0, 0)),
            pl.BlockSpec((2 * D, D), lambda j: (0, 0)),
            pl.BlockSpec((1, D), lambda j: (0, 0)),
        ],
        out_specs=[
            pl.BlockSpec((BLK, D), lambda j: (j, 0)),
            pl.BlockSpec((G, D), lambda j: (0, 0)),
        ],
        out_shape=[
            jax.ShapeDtypeStruct((N, D), jnp.float32),
            jax.ShapeDtypeStruct((G, D), jnp.float32),
        ],
        compiler_params=pltpu.CompilerParams(
            dimension_semantics=("arbitrary",)),
    )(X2, S2, batch, S1, SS, gamma2, beta2, Wh, bh)


def kernel(x, edge_index, batch, s, W1, W2, gamma, beta, Wg, bg, Wh, bh):
    src = edge_index[0].astype(jnp.int32)
    dst = edge_index[1].astype(jnp.int32)
    srcw = jnp.concatenate(
        [src, jnp.zeros((E_PAD - E,), jnp.int32)]).reshape(NTILES, NWIN, WIN)
    dstw = jnp.concatenate(
        [dst, jnp.full((E_PAD - E,), N_PAD - 1, jnp.int32)]
    ).reshape(NTILES, NWIN, WIN)
    dstd = jnp.concatenate(
        [dst, jnp.full((E_DEG_PAD - E,), N_PAD - 1, jnp.int32)]
    ).reshape(NCORES, NTILES, NWIN_DEG, WIN)

    degout = _sc_deg(dstd)
    Y, SWP, dinv = _tc_layer0(x, s, degout, W1[0], Wg[0])
    X2 = S2 = S1 = SS = None
    for i in range(L):
        agg = _sc_segsum(Y, SWP, srcw, dstw)
        if i < L - 1:
            Y, SWP = _tc_mid(Y, SWP, agg, dinv, W2[i], bg[i].reshape(1, D),
                             W1[i + 1], Wg[i + 1])
        else:
            X2, S2, S1, SS = _tc_last(Y, SWP, agg, dinv, W2[i],
                                      bg[i].reshape(1, D))
    x_local, pooled = _tc_final(X2, S2, batch.astype(jnp.int32).reshape(N, 1),
                                S1, SS, gamma[L - 1].reshape(1, D),
                                beta[L - 1].reshape(1, D), Wh,
                                bh.reshape(1, D))
    return (pooled, x_local)
